# Initial kernel scaffold; baseline (speedup 1.0000x reference)
import jax
import jax.numpy as jnp
from jax.experimental import pallas as pl


def kernel(x, edge_index, edge_attr, W_e, b_e, W_n, b_n):
    raise NotImplementedError("write your pallas kernel here")


# speedup vs baseline: 3.5768x; 3.5768x over previous
import functools

import jax
import jax.numpy as jnp
from jax import lax
from jax.experimental import pallas as pl
from jax.experimental.pallas import tpu as pltpu
from jax.experimental.pallas import tpu_sc as plsc

N = 10000
E = 320000
D = 128
DE = 16

NC = 2
NS = 16
NW = NC * NS
EPT = E // NW
B = 80
NB = EPT // B
NP = 10240
RPT = NP // NS


def _p12_body(x_ref, w1_ref, w2_ref, p1_ref, p2_ref):
    xb = x_ref[...]
    p1_ref[...] = jnp.dot(xb, w1_ref[...], preferred_element_type=jnp.float32)
    p2_ref[...] = jnp.dot(xb, w2_ref[...], preferred_element_type=jnp.float32)


def _q_body(ea_ref, w3_ref, be_ref, q_ref):
    q_ref[...] = (
        jnp.dot(ea_ref[...], w3_ref[...], preferred_element_type=jnp.float32)
        + be_ref[...]
    )


def _node_body(a0_ref, a1_ref, x_ref, wn1_ref, wn2_ref, bn_ref, o_ref):
    agg = a0_ref[...] + a1_ref[...]
    o_ref[...] = (
        jnp.dot(agg, wn1_ref[...], preferred_element_type=jnp.float32)
        + jnp.dot(x_ref[...], wn2_ref[...], preferred_element_type=jnp.float32)
        + bn_ref[...]
    )


def _sc_body(p1_h, p2_h, q_h, src_h, dst_h, zeros_h, eh_h, agg_h,
             srcb, dstb, r1, r2, qb, ehb, agg_s, s1, s2, s3):
    cid = lax.axis_index("c")
    sid = lax.axis_index("s")
    wid = sid * NC + cid
    base = wid * EPT

    pltpu.sync_copy(zeros_h.at[pl.ds(sid * RPT, RPT)],
                    agg_s.at[pl.ds(sid * RPT, RPT)])
    plsc.subcore_barrier()

    def block(j, carry):
        g = base + j * B
        pltpu.sync_copy(src_h.at[pl.ds(g, B)], srcb)
        pltpu.sync_copy(dst_h.at[pl.ds(g, B)], dstb)
        c1 = pltpu.async_copy(p1_h.at[srcb], r1, s1)
        c2 = pltpu.async_copy(p2_h.at[dstb], r2, s2)
        c3 = pltpu.async_copy(q_h.at[pl.ds(g, B)], qb, s3)
        c1.wait()
        c2.wait()
        c3.wait()

        def row(i, c):
            ehb[i] = r1[i] + r2[i] + qb[i]
            return c
        lax.fori_loop(0, B, row, 0, unroll=4)

        pltpu.sync_copy(ehb, eh_h.at[pl.ds(g, B)])
        pltpu.sync_copy(ehb, agg_s.at[dstb], add=True)
        return carry

    lax.fori_loop(0, NB, block, 0)

    plsc.subcore_barrier()
    pltpu.sync_copy(agg_s.at[pl.ds(sid * RPT, RPT)],
                    agg_h.at[cid, pl.ds(sid * RPT, RPT)])


@jax.jit
def kernel(x, edge_index, edge_attr, W_e, b_e, W_n, b_n):
    we1 = W_e[:D]
    we2 = W_e[D:2 * D]
    we3 = W_e[2 * D:]

    nblk = 10
    nrows = N // nblk
    p1, p2 = pl.pallas_call(
        _p12_body,
        grid=(nblk,),
        in_specs=[
            pl.BlockSpec((nrows, D), lambda i: (i, 0)),
            pl.BlockSpec((D, DE), lambda i: (0, 0)),
            pl.BlockSpec((D, DE), lambda i: (0, 0)),
        ],
        out_specs=[
            pl.BlockSpec((nrows, DE), lambda i: (i, 0)),
            pl.BlockSpec((nrows, DE), lambda i: (i, 0)),
        ],
        out_shape=[
            jax.ShapeDtypeStruct((N, DE), jnp.float32),
            jax.ShapeDtypeStruct((N, DE), jnp.float32),
        ],
    )(x, we1, we2)

    eblk = 16
    erows = E // eblk
    q = pl.pallas_call(
        _q_body,
        grid=(eblk,),
        in_specs=[
            pl.BlockSpec((erows, DE), lambda i: (i, 0)),
            pl.BlockSpec((DE, DE), lambda i: (0, 0)),
            pl.BlockSpec((1, DE), lambda i: (0, 0)),
        ],
        out_specs=pl.BlockSpec((erows, DE), lambda i: (i, 0)),
        out_shape=jax.ShapeDtypeStruct((E, DE), jnp.float32),
    )(edge_attr, we3, b_e.reshape(1, DE))

    zeros = jnp.zeros((NP, DE), jnp.float32)
    mesh = plsc.VectorSubcoreMesh(core_axis_name="c", subcore_axis_name="s")
    sc = pl.kernel(
        _sc_body,
        out_type=[
            jax.ShapeDtypeStruct((E, DE), jnp.float32),
            jax.ShapeDtypeStruct((NC, NP, DE), jnp.float32),
        ],
        mesh=mesh,
        compiler_params=pltpu.CompilerParams(use_tc_tiling_on_sc=False),
        scratch_types=[
            pltpu.VMEM((B,), jnp.int32),
            pltpu.VMEM((B,), jnp.int32),
            pltpu.VMEM((B, DE), jnp.float32),
            pltpu.VMEM((B, DE), jnp.float32),
            pltpu.VMEM((B, DE), jnp.float32),
            pltpu.VMEM((B, DE), jnp.float32),
            pltpu.VMEM_SHARED((NP, DE), jnp.float32),
            pltpu.SemaphoreType.DMA,
            pltpu.SemaphoreType.DMA,
            pltpu.SemaphoreType.DMA,
        ],
    )
    e_h, agg2 = sc(p1, p2, q, edge_index[0], edge_index[1], zeros)

    wn1 = W_n[:DE]
    wn2 = W_n[DE:]
    new_x = pl.pallas_call(
        _node_body,
        grid=(nblk,),
        in_specs=[
            pl.BlockSpec((nrows, DE), lambda i: (i, 0)),
            pl.BlockSpec((nrows, DE), lambda i: (i, 0)),
            pl.BlockSpec((nrows, D), lambda i: (i, 0)),
            pl.BlockSpec((DE, D), lambda i: (0, 0)),
            pl.BlockSpec((D, D), lambda i: (0, 0)),
            pl.BlockSpec((1, D), lambda i: (0, 0)),
        ],
        out_specs=pl.BlockSpec((nrows, D), lambda i: (i, 0)),
        out_shape=jax.ShapeDtypeStruct((N, D), jnp.float32),
    )(agg2[0], agg2[1], x, wn1, wn2, b_n.reshape(1, D))

    return new_x, e_h


# speedup vs baseline: 6.0100x; 1.6803x over previous
import jax
import jax.numpy as jnp
from jax import lax
from jax.experimental import pallas as pl
from jax.experimental.pallas import tpu as pltpu
from jax.experimental.pallas import tpu_sc as plsc

N = 10000
E = 320000
D = 128
DE = 16

NC = 2
NS = 16
NW = NC * NS
EPT = E // NW
B = 80
NB = EPT // B
BP = B + 1
NP = 10240
RPT = NP // NS


def _p12_body(x_ref, w1_ref, w2_ref, p1_ref, p2_ref):
    xb = x_ref[...]
    p1_ref[...] = jnp.dot(xb, w1_ref[...], preferred_element_type=jnp.float32)
    p2_ref[...] = jnp.dot(xb, w2_ref[...], preferred_element_type=jnp.float32)


def _qt_body(eat_ref, w3t_ref, bec_ref, qt_ref):
    qt_ref[...] = (
        jnp.dot(w3t_ref[...], eat_ref[...], preferred_element_type=jnp.float32)
        + bec_ref[...]
    )


def _node_body(a0_ref, a1_ref, x_ref, wn1_ref, wn2_ref, bn_ref, o_ref):
    agg = a0_ref[...] + a1_ref[...]
    o_ref[...] = (
        jnp.dot(agg, wn1_ref[...], preferred_element_type=jnp.float32)
        + jnp.dot(x_ref[...], wn2_ref[...], preferred_element_type=jnp.float32)
        + bn_ref[...]
    )


def _sc_body(p1_h, p2_h, qt_h, src_h, dst_h, zeros_h, ehT_h, agg_h,
             srcb, dstb, r1, r2, qtb, ehb, ehTb, agg_s, s1, s2, s3):
    cid = lax.axis_index("c")
    sid = lax.axis_index("s")
    wid = sid * NC + cid
    base = wid * EPT

    pltpu.sync_copy(zeros_h.at[pl.ds(sid * RPT, RPT)],
                    agg_s.at[pl.ds(sid * RPT, RPT)])
    plsc.subcore_barrier()

    iota = lax.iota(jnp.int32, 16)

    def block(j, carry):
        g = base + j * B
        pltpu.sync_copy(src_h.at[pl.ds(g, B)], srcb)
        pltpu.sync_copy(dst_h.at[pl.ds(g, B)], dstb)
        c1 = pltpu.async_copy(p1_h.at[srcb], r1, s1)
        c2 = pltpu.async_copy(p2_h.at[dstb], r2, s2)
        c3 = pltpu.async_copy(qt_h.at[:, pl.ds(g, B)], qtb.at[:, pl.ds(0, B)],
                              s3)
        c1.wait()
        c2.wait()
        c3.wait()

        def row(i, c):
            col = jnp.full((16,), i, jnp.int32)
            qcol = plsc.load_gather(qtb, [iota, col])
            e = r1[i] + r2[i] + qcol
            ehb[i] = e
            plsc.store_scatter(ehTb, [iota, col], e)
            return c
        lax.fori_loop(0, B, row, 0, unroll=4)

        pltpu.sync_copy(ehTb.at[:, pl.ds(0, B)], ehT_h.at[:, pl.ds(g, B)])
        pltpu.sync_copy(ehb, agg_s.at[dstb], add=True)
        return carry

    lax.fori_loop(0, NB, block, 0)

    plsc.subcore_barrier()
    pltpu.sync_copy(agg_s.at[pl.ds(sid * RPT, RPT)],
                    agg_h.at[cid, pl.ds(sid * RPT, RPT)])


@jax.jit
def kernel(x, edge_index, edge_attr, W_e, b_e, W_n, b_n):
    we1 = W_e[:D]
    we2 = W_e[D:2 * D]
    we3t = W_e[2 * D:].T

    nblk = 10
    nrows = N // nblk
    p1, p2 = pl.pallas_call(
        _p12_body,
        grid=(nblk,),
        in_specs=[
            pl.BlockSpec((nrows, D), lambda i: (i, 0)),
            pl.BlockSpec((D, DE), lambda i: (0, 0)),
            pl.BlockSpec((D, DE), lambda i: (0, 0)),
        ],
        out_specs=[
            pl.BlockSpec((nrows, DE), lambda i: (i, 0)),
            pl.BlockSpec((nrows, DE), lambda i: (i, 0)),
        ],
        out_shape=[
            jax.ShapeDtypeStruct((N, DE), jnp.float32),
            jax.ShapeDtypeStruct((N, DE), jnp.float32),
        ],
    )(x, we1, we2)

    eblk = 50
    ecols = E // eblk
    qt = pl.pallas_call(
        _qt_body,
        grid=(eblk,),
        in_specs=[
            pl.BlockSpec((DE, ecols), lambda i: (0, i)),
            pl.BlockSpec((DE, DE), lambda i: (0, 0)),
            pl.BlockSpec((DE, 1), lambda i: (0, 0)),
        ],
        out_specs=pl.BlockSpec((DE, ecols), lambda i: (0, i)),
        out_shape=jax.ShapeDtypeStruct((DE, E), jnp.float32),
    )(edge_attr.T, we3t, b_e.reshape(DE, 1))

    zeros = jnp.zeros((NP, DE), jnp.float32)
    mesh = plsc.VectorSubcoreMesh(core_axis_name="c", subcore_axis_name="s")
    sc = pl.kernel(
        _sc_body,
        out_type=[
            jax.ShapeDtypeStruct((DE, E), jnp.float32),
            jax.ShapeDtypeStruct((NC, NP, DE), jnp.float32),
        ],
        mesh=mesh,
        compiler_params=pltpu.CompilerParams(use_tc_tiling_on_sc=False,
                                             needs_layout_passes=False),
        scratch_types=[
            pltpu.VMEM((B,), jnp.int32),
            pltpu.VMEM((B,), jnp.int32),
            pltpu.VMEM((B, DE), jnp.float32),
            pltpu.VMEM((B, DE), jnp.float32),
            pltpu.VMEM((DE, BP), jnp.float32),
            pltpu.VMEM((B, DE), jnp.float32),
            pltpu.VMEM((DE, BP), jnp.float32),
            pltpu.VMEM_SHARED((NP, DE), jnp.float32),
            pltpu.SemaphoreType.DMA,
            pltpu.SemaphoreType.DMA,
            pltpu.SemaphoreType.DMA,
        ],
    )
    ehT, agg2 = sc(p1, p2, qt, edge_index[0], edge_index[1], zeros)

    wn1 = W_n[:DE]
    wn2 = W_n[DE:]
    new_x = pl.pallas_call(
        _node_body,
        grid=(nblk,),
        in_specs=[
            pl.BlockSpec((nrows, DE), lambda i: (i, 0)),
            pl.BlockSpec((nrows, DE), lambda i: (i, 0)),
            pl.BlockSpec((nrows, D), lambda i: (i, 0)),
            pl.BlockSpec((DE, D), lambda i: (0, 0)),
            pl.BlockSpec((D, D), lambda i: (0, 0)),
            pl.BlockSpec((1, D), lambda i: (0, 0)),
        ],
        out_specs=pl.BlockSpec((nrows, D), lambda i: (i, 0)),
        out_shape=jax.ShapeDtypeStruct((N, D), jnp.float32),
    )(agg2[0], agg2[1], x, wn1, wn2, b_n.reshape(1, D))

    return new_x, ehT.T


# speedup vs baseline: 9.4556x; 1.5733x over previous
import jax
import jax.numpy as jnp
from jax import lax
from jax.experimental import pallas as pl
from jax.experimental.pallas import tpu as pltpu
from jax.experimental.pallas import tpu_sc as plsc

N = 10000
E = 320000
D = 128
DE = 16

NC = 2
NS = 16
NW = NC * NS
EPT = E // NW
B = 80
NB = EPT // B
BP = B + 1
NP = 10240
RPT = NP // NS


def _p12_body(x_ref, w1_ref, w2_ref, p1_ref, p2_ref):
    xb = x_ref[...]
    p1_ref[...] = jnp.dot(xb, w1_ref[...], preferred_element_type=jnp.float32)
    p2_ref[...] = jnp.dot(xb, w2_ref[...], preferred_element_type=jnp.float32)


def _qt_body(eat_ref, w3t_ref, bec_ref, qt_ref):
    qt_ref[...] = (
        jnp.dot(w3t_ref[...], eat_ref[...], preferred_element_type=jnp.float32)
        + bec_ref[...]
    )


def _node_body(a0_ref, a1_ref, x_ref, wn1_ref, wn2_ref, bn_ref, o_ref):
    agg = a0_ref[...] + a1_ref[...]
    o_ref[...] = (
        jnp.dot(agg, wn1_ref[...], preferred_element_type=jnp.float32)
        + jnp.dot(x_ref[...], wn2_ref[...], preferred_element_type=jnp.float32)
        + bn_ref[...]
    )


def _sc_body(p1_h, p2_h, qt_h, src_h, dst_h, zeros_h, ehT_h, agg_h,
             src_all, dst_all, r1, r2, qtb, ehb, ehTb, agg_s, s_gat):
    cid = lax.axis_index("c")
    sid = lax.axis_index("s")
    wid = sid * NC + cid
    base = wid * EPT

    pltpu.sync_copy(zeros_h.at[pl.ds(sid * RPT, RPT)],
                    agg_s.at[pl.ds(sid * RPT, RPT)])

    pltpu.sync_copy(src_h.at[pl.ds(wid * NB, NB)], src_all)
    pltpu.sync_copy(dst_h.at[pl.ds(wid * NB, NB)], dst_all)

    iota = lax.iota(jnp.int32, 16)

    def gat_copies(j):
        g = base + j * B
        k = j % 2
        return (
            pltpu.make_async_copy(p1_h.at[src_all.at[j]], r1.at[k], s_gat),
            pltpu.make_async_copy(p2_h.at[dst_all.at[j]], r2.at[k], s_gat),
            pltpu.make_async_copy(qt_h.at[:, pl.ds(g, B)],
                                  qtb.at[k, :, pl.ds(0, B)], s_gat),
        )

    for c in gat_copies(0):
        c.start()
    for c in gat_copies(1):
        c.start()

    plsc.subcore_barrier()

    def block(j, carry):
        k = j % 2
        for c in gat_copies(j):
            c.wait()

        def row(i, c):
            col = jnp.full((16,), i, jnp.int32)
            qcol = plsc.load_gather(qtb.at[k], [iota, col])
            e = r1[k, i] + r2[k, i] + qcol
            ehb[i] = e
            plsc.store_scatter(ehTb.at[k], [iota, col], e)
            return c
        lax.fori_loop(0, B, row, 0, unroll=8)

        @pl.when(j + 2 < NB)
        def _():
            for c in gat_copies(j + 2):
                c.start()

        g = base + j * B
        pltpu.sync_copy(ehTb.at[k, :, pl.ds(0, B)], ehT_h.at[:, pl.ds(g, B)])
        pltpu.sync_copy(ehb, agg_s.at[dst_all.at[j]], add=True)
        return carry

    lax.fori_loop(0, NB, block, 0)

    plsc.subcore_barrier()
    pltpu.sync_copy(agg_s.at[pl.ds(sid * RPT, RPT)],
                    agg_h.at[cid, pl.ds(sid * RPT, RPT)])


@jax.jit
def kernel(x, edge_index, edge_attr, W_e, b_e, W_n, b_n):
    we1 = W_e[:D]
    we2 = W_e[D:2 * D]
    we3t = W_e[2 * D:].T

    nblk = 10
    nrows = N // nblk
    p1, p2 = pl.pallas_call(
        _p12_body,
        grid=(nblk,),
        in_specs=[
            pl.BlockSpec((nrows, D), lambda i: (i, 0)),
            pl.BlockSpec((D, DE), lambda i: (0, 0)),
            pl.BlockSpec((D, DE), lambda i: (0, 0)),
        ],
        out_specs=[
            pl.BlockSpec((nrows, DE), lambda i: (i, 0)),
            pl.BlockSpec((nrows, DE), lambda i: (i, 0)),
        ],
        out_shape=[
            jax.ShapeDtypeStruct((N, DE), jnp.float32),
            jax.ShapeDtypeStruct((N, DE), jnp.float32),
        ],
    )(x, we1, we2)

    eblk = 50
    ecols = E // eblk
    qt = pl.pallas_call(
        _qt_body,
        grid=(eblk,),
        in_specs=[
            pl.BlockSpec((DE, ecols), lambda i: (0, i)),
            pl.BlockSpec((DE, DE), lambda i: (0, 0)),
            pl.BlockSpec((DE, 1), lambda i: (0, 0)),
        ],
        out_specs=pl.BlockSpec((DE, ecols), lambda i: (0, i)),
        out_shape=jax.ShapeDtypeStruct((DE, E), jnp.float32),
    )(edge_attr.T, we3t, b_e.reshape(DE, 1))

    zeros = jnp.zeros((NP, DE), jnp.float32)
    mesh = plsc.VectorSubcoreMesh(core_axis_name="c", subcore_axis_name="s")
    sc = pl.kernel(
        _sc_body,
        out_type=[
            jax.ShapeDtypeStruct((DE, E), jnp.float32),
            jax.ShapeDtypeStruct((NC, NP, DE), jnp.float32),
        ],
        mesh=mesh,
        compiler_params=pltpu.CompilerParams(use_tc_tiling_on_sc=False,
                                             needs_layout_passes=False),
        scratch_types=[
            pltpu.VMEM((NB, B), jnp.int32),
            pltpu.VMEM((NB, B), jnp.int32),
            pltpu.VMEM((2, B, DE), jnp.float32),
            pltpu.VMEM((2, B, DE), jnp.float32),
            pltpu.VMEM((2, DE, BP), jnp.float32),
            pltpu.VMEM((B, DE), jnp.float32),
            pltpu.VMEM((2, DE, BP), jnp.float32),
            pltpu.VMEM_SHARED((NP, DE), jnp.float32),
            pltpu.SemaphoreType.DMA,
        ],
    )
    ehT, agg2 = sc(p1, p2, qt,
                   edge_index[0].reshape(E // B, B),
                   edge_index[1].reshape(E // B, B), zeros)

    wn1 = W_n[:DE]
    wn2 = W_n[DE:]
    new_x = pl.pallas_call(
        _node_body,
        grid=(nblk,),
        in_specs=[
            pl.BlockSpec((nrows, DE), lambda i: (i, 0)),
            pl.BlockSpec((nrows, DE), lambda i: (i, 0)),
            pl.BlockSpec((nrows, D), lambda i: (i, 0)),
            pl.BlockSpec((DE, D), lambda i: (0, 0)),
            pl.BlockSpec((D, D), lambda i: (0, 0)),
            pl.BlockSpec((1, D), lambda i: (0, 0)),
        ],
        out_specs=pl.BlockSpec((nrows, D), lambda i: (i, 0)),
        out_shape=jax.ShapeDtypeStruct((N, D), jnp.float32),
    )(agg2[0], agg2[1], x, wn1, wn2, b_n.reshape(1, D))

    return new_x, ehT.T


# speedup vs baseline: 9.7098x; 1.0269x over previous
import jax
import jax.numpy as jnp
from jax import lax
from jax.experimental import pallas as pl
from jax.experimental.pallas import tpu as pltpu
from jax.experimental.pallas import tpu_sc as plsc

N = 10000
E = 320000
D = 128
DE = 16

NC = 2
NS = 16
NW = NC * NS
EPT = E // NW
B = 80
NB = EPT // B
BP = B + 1
NP = 10240
RPT = NP // NS


def _p12_body(x_ref, w1_ref, w2_ref, p1_ref, p2_ref):
    xb = x_ref[...]
    p1_ref[...] = jnp.dot(xb, w1_ref[...], preferred_element_type=jnp.float32)
    p2_ref[...] = jnp.dot(xb, w2_ref[...], preferred_element_type=jnp.float32)


def _qt_body(eat_ref, w3t_ref, bec_ref, qt_ref):
    qt_ref[...] = (
        jnp.dot(w3t_ref[...], eat_ref[...], preferred_element_type=jnp.float32)
        + bec_ref[...]
    )


def _node_body(a0_ref, a1_ref, x_ref, wn1_ref, wn2_ref, bn_ref, o_ref):
    agg = a0_ref[...] + a1_ref[...]
    o_ref[...] = (
        jnp.dot(agg, wn1_ref[...], preferred_element_type=jnp.float32)
        + jnp.dot(x_ref[...], wn2_ref[...], preferred_element_type=jnp.float32)
        + bn_ref[...]
    )


def _sc_body(p1_h, p2_h, qt_h, src_h, dst_h, zeros_h, ehT_h, agg_h,
             src_all, dst_all, r1, r2, qtb, ehb, ehTb, agg_s, s_gat, s_wr):
    cid = lax.axis_index("c")
    sid = lax.axis_index("s")
    wid = sid * NC + cid
    base = wid * EPT

    pltpu.sync_copy(zeros_h.at[pl.ds(sid * RPT, RPT)],
                    agg_s.at[pl.ds(sid * RPT, RPT)])

    pltpu.sync_copy(src_h.at[pl.ds(wid * NB, NB)], src_all)
    pltpu.sync_copy(dst_h.at[pl.ds(wid * NB, NB)], dst_all)

    iota = lax.iota(jnp.int32, 16)

    def gat_copies(j):
        g = base + j * B
        k = j % 2
        return (
            pltpu.make_async_copy(p1_h.at[src_all.at[j]], r1.at[k], s_gat),
            pltpu.make_async_copy(p2_h.at[dst_all.at[j]], r2.at[k], s_gat),
            pltpu.make_async_copy(qt_h.at[:, pl.ds(g, B)],
                                  qtb.at[k, :, pl.ds(0, B)], s_gat),
        )

    def ehT_copy(j):
        g = base + j * B
        k = j % 2
        return pltpu.make_async_copy(ehTb.at[k, :, pl.ds(0, B)],
                                     ehT_h.at[:, pl.ds(g, B)], s_wr)

    for c in gat_copies(0):
        c.start()
    for c in gat_copies(1):
        c.start()

    plsc.subcore_barrier()

    def block(j, carry):
        k = j % 2
        for c in gat_copies(j):
            c.wait()

        @pl.when(j >= 2)
        def _():
            ehT_copy(j - 2).wait()

        def row(i, c):
            col = jnp.full((16,), i, jnp.int32)
            qcol = plsc.load_gather(qtb.at[k], [iota, col])
            e = r1[k, i] + r2[k, i] + qcol
            ehb[i] = e
            plsc.store_scatter(ehTb.at[k], [iota, col], e)
            return c
        lax.fori_loop(0, B, row, 0, unroll=8)

        ehT_copy(j).start()

        @pl.when(j + 2 < NB)
        def _():
            for c in gat_copies(j + 2):
                c.start()

        pltpu.sync_copy(ehb, agg_s.at[dst_all.at[j]], add=True)
        return carry

    lax.fori_loop(0, NB, block, 0)

    ehT_copy(NB - 2).wait()
    ehT_copy(NB - 1).wait()

    plsc.subcore_barrier()
    pltpu.sync_copy(agg_s.at[pl.ds(sid * RPT, RPT)],
                    agg_h.at[cid, pl.ds(sid * RPT, RPT)])


@jax.jit
def kernel(x, edge_index, edge_attr, W_e, b_e, W_n, b_n):
    we1 = W_e[:D]
    we2 = W_e[D:2 * D]
    we3t = W_e[2 * D:].T

    nblk = 10
    nrows = N // nblk
    p1, p2 = pl.pallas_call(
        _p12_body,
        grid=(nblk,),
        in_specs=[
            pl.BlockSpec((nrows, D), lambda i: (i, 0)),
            pl.BlockSpec((D, DE), lambda i: (0, 0)),
            pl.BlockSpec((D, DE), lambda i: (0, 0)),
        ],
        out_specs=[
            pl.BlockSpec((nrows, DE), lambda i: (i, 0)),
            pl.BlockSpec((nrows, DE), lambda i: (i, 0)),
        ],
        out_shape=[
            jax.ShapeDtypeStruct((N, DE), jnp.float32),
            jax.ShapeDtypeStruct((N, DE), jnp.float32),
        ],
    )(x, we1, we2)

    eblk = 50
    ecols = E // eblk
    qt = pl.pallas_call(
        _qt_body,
        grid=(eblk,),
        in_specs=[
            pl.BlockSpec((DE, ecols), lambda i: (0, i)),
            pl.BlockSpec((DE, DE), lambda i: (0, 0)),
            pl.BlockSpec((DE, 1), lambda i: (0, 0)),
        ],
        out_specs=pl.BlockSpec((DE, ecols), lambda i: (0, i)),
        out_shape=jax.ShapeDtypeStruct((DE, E), jnp.float32),
    )(edge_attr.T, we3t, b_e.reshape(DE, 1))

    zeros = jnp.zeros((NP, DE), jnp.float32)
    mesh = plsc.VectorSubcoreMesh(core_axis_name="c", subcore_axis_name="s")
    sc = pl.kernel(
        _sc_body,
        out_type=[
            jax.ShapeDtypeStruct((DE, E), jnp.float32),
            jax.ShapeDtypeStruct((NC, NP, DE), jnp.float32),
        ],
        mesh=mesh,
        compiler_params=pltpu.CompilerParams(use_tc_tiling_on_sc=False,
                                             needs_layout_passes=False),
        scratch_types=[
            pltpu.VMEM((NB, B), jnp.int32),
            pltpu.VMEM((NB, B), jnp.int32),
            pltpu.VMEM((2, B, DE), jnp.float32),
            pltpu.VMEM((2, B, DE), jnp.float32),
            pltpu.VMEM((2, DE, BP), jnp.float32),
            pltpu.VMEM((B, DE), jnp.float32),
            pltpu.VMEM((2, DE, BP), jnp.float32),
            pltpu.VMEM_SHARED((NP, DE), jnp.float32),
            pltpu.SemaphoreType.DMA,
            pltpu.SemaphoreType.DMA,
        ],
    )
    ehT, agg2 = sc(p1, p2, qt,
                   edge_index[0].reshape(E // B, B),
                   edge_index[1].reshape(E // B, B), zeros)

    wn1 = W_n[:DE]
    wn2 = W_n[DE:]
    new_x = pl.pallas_call(
        _node_body,
        grid=(nblk,),
        in_specs=[
            pl.BlockSpec((nrows, DE), lambda i: (i, 0)),
            pl.BlockSpec((nrows, DE), lambda i: (i, 0)),
            pl.BlockSpec((nrows, D), lambda i: (i, 0)),
            pl.BlockSpec((DE, D), lambda i: (0, 0)),
            pl.BlockSpec((D, D), lambda i: (0, 0)),
            pl.BlockSpec((1, D), lambda i: (0, 0)),
        ],
        out_specs=pl.BlockSpec((nrows, D), lambda i: (i, 0)),
        out_shape=jax.ShapeDtypeStruct((N, D), jnp.float32),
    )(agg2[0], agg2[1], x, wn1, wn2, b_n.reshape(1, D))

    return new_x, ehT.T
